# baseline (device time: 67168 ns/iter reference)
import jax
import jax.numpy as jnp
from jax import lax
from jax.experimental import pallas as pl
from jax.experimental.pallas import tpu as pltpu

N_DEV = 8
ORDERS = ((0, 1, 2), (1, 2, 0), (2, 0, 1))
COL_OFF = (0, 768, 1408)
COL_W = (768, 640, 640)
N_VAR = len(ORDERS)

RELS = [tuple((r >> a) & 1 for a in range(3)) for r in range(8)]


def _rint(r):
    return r[0] + 2 * r[1] + 4 * r[2]


def _parent_rel(r, order):
    for a in order:
        if r[a]:
            p = list(r)
            p[a] = 0
            return tuple(p), a
    return None, None


def _children_rel(r, order):
    out = []
    for a in order:
        if r[a]:
            break
        c = list(r)
        c[a] = 1
        out.append((tuple(c), a))
    return out


RECV_SLOT = []
ROLE_SEM = []
for order in ORDERS:
    pairs = []
    sems = {}
    for r in RELS:
        kids = _children_rel(r, order)
        if kids:
            sems[r] = len(sems)
        for _c, a in kids:
            pairs.append((r, a))
    RECV_SLOT.append({pa: i for i, pa in enumerate(pairs)})
    ROLE_SEM.append(sems)

def _roles():
    leaves, interior = [], []
    for pc in (3, 2, 1):
        for t, order in enumerate(ORDERS):
            for r in RELS:
                if sum(r) != pc:
                    continue
                (leaves if not _children_rel(r, order) else interior).append(
                    (pc, t, r))
    key = lambda role: (-role[0], role[2], role[1])
    leaves.sort(key=key)
    interior.sort(key=key)
    return leaves, interior


LEAF_ROLES, INTERIOR_ROLES = _roles()


def _gelu(y):
    c = 0.7978845608028654
    return 0.5 * y * (1.0 + jnp.tanh(c * (y + 0.044715 * y * y * y)))


def kernel(x, w_mat):
    m_total, k_shard = x.shape
    _, n = w_mat.shape
    m_per = m_total // N_DEV

    def body(x_ref, w_ref, out_ref, *scratch):
        acc = scratch[0:N_VAR]
        rcv = scratch[N_VAR:2 * N_VAR]
        ss = scratch[2 * N_VAR:3 * N_VAR]
        rs = scratch[3 * N_VAR:4 * N_VAR]

        my = lax.axis_index("i")
        mz = lax.div(my, 4)
        rr = lax.rem(my, 4)
        mjy = lax.div(rr, 2)
        mx = lax.rem(lax.rem(rr, 2) + mjy, 2)
        mc = (mx, mjy, mz)

        def pos_of(c):
            return 4 * c[2] + 2 * c[1] + lax.rem(c[0] + c[1], 2)

        def nbr_dev(a):
            f = list(mc)
            f[a] = 1 - f[a]
            return pos_of(f)

        def root_chunk(r):
            c = [(1 - mc[a]) if r[a] else mc[a] for a in range(3)]
            return pos_of(c)

        barrier_sem = pltpu.get_barrier_semaphore()
        for a in range(3):
            pl.semaphore_signal(
                barrier_sem, inc=1,
                device_id=(nbr_dev(a),), device_id_type=pl.DeviceIdType.MESH,
            )
        pl.semaphore_wait(barrier_sem, 3)

        def dot(c, t):
            return jnp.dot(
                x_ref[pl.ds(c * m_per, m_per), :],
                w_ref[:, COL_OFF[t]:COL_OFF[t] + COL_W[t]],
                preferred_element_type=jnp.float32,
            )

        def send_desc(t, r):
            p, a = _parent_rel(r, ORDERS[t])
            slot = RECV_SLOT[t][(p, a)]
            return pltpu.make_async_remote_copy(
                src_ref=acc[t].at[_rint(r)],
                dst_ref=rcv[t].at[slot],
                send_sem=ss[t].at[0],
                recv_sem=rs[t].at[ROLE_SEM[t][p]],
                device_id=(nbr_dev(a),),
                device_id_type=pl.DeviceIdType.MESH,
            )

        def wait_children(t, r):
            for _crel, a in _children_rel(r, ORDERS[t]):
                slot = RECV_SLOT[t][(r, a)]
                pltpu.make_async_remote_copy(
                    src_ref=acc[t].at[_rint(r)],
                    dst_ref=rcv[t].at[slot],
                    send_sem=ss[t].at[0],
                    recv_sem=rs[t].at[ROLE_SEM[t][r]],
                    device_id=(nbr_dev(a),),
                    device_id_type=pl.DeviceIdType.MESH,
                ).wait_recv()

        started = []
        for _pc, t, r in LEAF_ROLES:
            ri = _rint(r)
            acc[t][ri] = dot(root_chunk(r), t)
            send_desc(t, r).start()
            started.append((t, r))

        for t in range(N_VAR):
            acc[t][0] = dot(pos_of(mc), t)

        for _pc, t, r in INTERIOR_ROLES:
            ri = _rint(r)
            acc[t][ri] = dot(root_chunk(r), t)
            wait_children(t, r)
            for _crel, a in _children_rel(r, ORDERS[t]):
                slot = RECV_SLOT[t][(r, a)]
                acc[t][ri] = acc[t][ri] + rcv[t][slot]
            send_desc(t, r).start()
            started.append((t, r))

        for t in range(N_VAR):
            wait_children(t, (0, 0, 0))
            for _crel, a in _children_rel((0, 0, 0), ORDERS[t]):
                slot = RECV_SLOT[t][((0, 0, 0), a)]
                acc[t][0] = acc[t][0] + rcv[t][slot]
            out_ref[:, COL_OFF[t]:COL_OFF[t] + COL_W[t]] = _gelu(acc[t][0])

        for t, r in started:
            send_desc(t, r).wait_send()

    return pl.pallas_call(
        body,
        out_shape=jax.ShapeDtypeStruct((m_per, n), jnp.float32),
        in_specs=[
            pl.BlockSpec(memory_space=pltpu.VMEM),
            pl.BlockSpec(memory_space=pltpu.VMEM),
        ],
        out_specs=pl.BlockSpec(memory_space=pltpu.VMEM),
        scratch_shapes=(
            [pltpu.VMEM((N_DEV, m_per, COL_W[t]), jnp.float32)
             for t in range(N_VAR)]
            + [pltpu.VMEM((7, m_per, COL_W[t]), jnp.float32)
               for t in range(N_VAR)]
            + [pltpu.SemaphoreType.DMA((1,)) for _ in range(N_VAR)]
            + [pltpu.SemaphoreType.DMA((len(ROLE_SEM[t]),))
               for t in range(N_VAR)]
        ),
        compiler_params=pltpu.CompilerParams(collective_id=0),
    )(x, w_mat)


# device time: 65747 ns/iter; 1.0216x vs baseline; 1.0216x over previous
import jax
import jax.numpy as jnp
from jax import lax
from jax.experimental import pallas as pl
from jax.experimental.pallas import tpu as pltpu

N_DEV = 8
ORDERS = ((0, 1, 2), (1, 2, 0), (2, 0, 1))
COL_OFF = (0, 768, 1408)
COL_W = (768, 640, 640)
N_VAR = len(ORDERS)

RELS = [tuple((r >> a) & 1 for a in range(3)) for r in range(8)]


def _rint(r):
    return r[0] + 2 * r[1] + 4 * r[2]


def _parent_rel(r, order):
    for a in order:
        if r[a]:
            p = list(r)
            p[a] = 0
            return tuple(p), a
    return None, None


def _children_rel(r, order):
    out = []
    for a in order:
        if r[a]:
            break
        c = list(r)
        c[a] = 1
        out.append((tuple(c), a))
    return out


RECV_SLOT = []
ROLE_SEM = []
for order in ORDERS:
    pairs = []
    sems = {}
    for r in RELS:
        kids = _children_rel(r, order)
        if kids:
            sems[r] = len(sems)
        for _c, a in kids:
            pairs.append((r, a))
    RECV_SLOT.append({pa: i for i, pa in enumerate(pairs)})
    ROLE_SEM.append(sems)

def _roles():
    leaves, interior = [], []
    for pc in (3, 2, 1):
        for t, order in enumerate(ORDERS):
            for r in RELS:
                if sum(r) != pc:
                    continue
                (leaves if not _children_rel(r, order) else interior).append(
                    (pc, t, r))
    key = lambda role: (-role[0], role[2], role[1])
    leaves.sort(key=key)
    ordered = {(t, r) for _pc, t, r in leaves}
    waves = []
    pending = interior[:]
    while pending:
        wave = [
            role for role in pending
            if all((role[1], c) in ordered
                   for c, _a in _children_rel(role[2], ORDERS[role[1]]))
        ]
        assert wave, "cyclic interior dependencies"
        wave.sort(key=key)
        waves.extend(wave)
        ordered |= {(t, r) for _pc, t, r in wave}
        pending = [x for x in pending if x not in wave]
    return leaves, waves


LEAF_ROLES, INTERIOR_ROLES = _roles()


def _gelu(y):
    c = 0.7978845608028654
    return 0.5 * y * (1.0 + jnp.tanh(c * (y + 0.044715 * y * y * y)))


def kernel(x, w_mat):
    m_total, k_shard = x.shape
    _, n = w_mat.shape
    m_per = m_total // N_DEV

    def body(x_ref, w_ref, out_ref, *scratch):
        acc = scratch[0:N_VAR]
        rcv = scratch[N_VAR:2 * N_VAR]
        ss = scratch[2 * N_VAR:3 * N_VAR]
        rs = scratch[3 * N_VAR:4 * N_VAR]

        my = lax.axis_index("i")
        mz = lax.div(my, 4)
        rr = lax.rem(my, 4)
        mjy = lax.div(rr, 2)
        mx = lax.rem(lax.rem(rr, 2) + mjy, 2)
        mc = (mx, mjy, mz)

        def pos_of(c):
            return 4 * c[2] + 2 * c[1] + lax.rem(c[0] + c[1], 2)

        def nbr_dev(a):
            f = list(mc)
            f[a] = 1 - f[a]
            return pos_of(f)

        def root_chunk(r):
            c = [(1 - mc[a]) if r[a] else mc[a] for a in range(3)]
            return pos_of(c)

        barrier_sem = pltpu.get_barrier_semaphore()
        for a in range(3):
            pl.semaphore_signal(
                barrier_sem, inc=1,
                device_id=(nbr_dev(a),), device_id_type=pl.DeviceIdType.MESH,
            )
        pl.semaphore_wait(barrier_sem, 3)

        def dot(c, t):
            return jnp.dot(
                x_ref[pl.ds(c * m_per, m_per), :],
                w_ref[:, COL_OFF[t]:COL_OFF[t] + COL_W[t]],
                preferred_element_type=jnp.float32,
            )

        def send_desc(t, r):
            p, a = _parent_rel(r, ORDERS[t])
            slot = RECV_SLOT[t][(p, a)]
            return pltpu.make_async_remote_copy(
                src_ref=acc[t].at[_rint(r)],
                dst_ref=rcv[t].at[slot],
                send_sem=ss[t].at[0],
                recv_sem=rs[t].at[ROLE_SEM[t][p]],
                device_id=(nbr_dev(a),),
                device_id_type=pl.DeviceIdType.MESH,
            )

        def wait_children(t, r):
            for _crel, a in _children_rel(r, ORDERS[t]):
                slot = RECV_SLOT[t][(r, a)]
                pltpu.make_async_remote_copy(
                    src_ref=acc[t].at[_rint(r)],
                    dst_ref=rcv[t].at[slot],
                    send_sem=ss[t].at[0],
                    recv_sem=rs[t].at[ROLE_SEM[t][r]],
                    device_id=(nbr_dev(a),),
                    device_id_type=pl.DeviceIdType.MESH,
                ).wait_recv()

        started = []
        for _pc, t, r in LEAF_ROLES:
            ri = _rint(r)
            acc[t][ri] = dot(root_chunk(r), t)
            send_desc(t, r).start()
            started.append((t, r))

        for t in range(N_VAR):
            acc[t][0] = dot(pos_of(mc), t)

        for _pc, t, r in INTERIOR_ROLES:
            ri = _rint(r)
            acc[t][ri] = dot(root_chunk(r), t)
            wait_children(t, r)
            for _crel, a in _children_rel(r, ORDERS[t]):
                slot = RECV_SLOT[t][(r, a)]
                acc[t][ri] = acc[t][ri] + rcv[t][slot]
            send_desc(t, r).start()
            started.append((t, r))

        for t in range(N_VAR):
            wait_children(t, (0, 0, 0))
            for _crel, a in _children_rel((0, 0, 0), ORDERS[t]):
                slot = RECV_SLOT[t][((0, 0, 0), a)]
                acc[t][0] = acc[t][0] + rcv[t][slot]
            out_ref[:, COL_OFF[t]:COL_OFF[t] + COL_W[t]] = _gelu(acc[t][0])

        for t, r in started:
            send_desc(t, r).wait_send()

    return pl.pallas_call(
        body,
        out_shape=jax.ShapeDtypeStruct((m_per, n), jnp.float32),
        in_specs=[
            pl.BlockSpec(memory_space=pltpu.VMEM),
            pl.BlockSpec(memory_space=pltpu.VMEM),
        ],
        out_specs=pl.BlockSpec(memory_space=pltpu.VMEM),
        scratch_shapes=(
            [pltpu.VMEM((N_DEV, m_per, COL_W[t]), jnp.float32)
             for t in range(N_VAR)]
            + [pltpu.VMEM((7, m_per, COL_W[t]), jnp.float32)
               for t in range(N_VAR)]
            + [pltpu.SemaphoreType.DMA((1,)) for _ in range(N_VAR)]
            + [pltpu.SemaphoreType.DMA((len(ROLE_SEM[t]),))
               for t in range(N_VAR)]
        ),
        compiler_params=pltpu.CompilerParams(collective_id=0),
    )(x, w_mat)
